# baseline (speedup 1.0000x reference)
import functools

import jax
import jax.numpy as jnp
import numpy as np
from jax import lax
from jax.experimental import pallas as pl
from jax.experimental.pallas import tpu as pltpu
from jax.experimental.pallas import tpu_sc as plsc

_SR = 24000
_IR_LEN = 1200
_N_DELAYS = 8
_EARLY_LEN = 200
_B = 4096

_C_EARLY = 0
_C_IN = 200
_C_OUT = 208
_C_GAP = 216
_C_DAP = 248
_C_T0 = 280
_C_ALPHA = 281
_DP = 384

_NC = 2
_NS = 16
_NW = _NC * _NS
_BPW = _B // _NW

_BB = 256
_GRID = _B // _BB


def _sc_gather(table, idx):
    mesh = plsc.VectorSubcoreMesh(core_axis_name="c", subcore_axis_name="s")

    @functools.partial(
        pl.kernel,
        mesh=mesh,
        out_type=jax.ShapeDtypeStruct((_B, _DP), jnp.float32),
        scratch_types=[
            pltpu.VMEM((_BPW,), jnp.int32),
            pltpu.VMEM((_BPW, _DP), jnp.float32),
            pltpu.SemaphoreType.DMA,
        ],
    )
    def k(table_hbm, idx_hbm, out_hbm, idx_v, rows_v, sem):
        wid = lax.axis_index("s") * _NC + lax.axis_index("c")
        base = wid * _BPW
        pltpu.sync_copy(idx_hbm.at[pl.ds(base, _BPW)], idx_v)
        pltpu.async_copy(table_hbm.at[idx_v], rows_v, sem).wait()
        pltpu.sync_copy(rows_v, out_hbm.at[pl.ds(base, _BPW)])

    return k(table, idx)


_CHUNK = 128
_NCHUNK = -(-_IR_LEN // _CHUNK)
_TPAD = _NCHUNK * _CHUNK

_PI_HI = 3.140625
_PI_LO = 9.6765358979e-04


def _sincos(x):
    m = jnp.floor(x * (1.0 / np.pi) + 0.5)
    xr = (x - m * _PI_HI) - m * _PI_LO
    sign = jnp.where((m.astype(jnp.int32) & 1) == 0, 1.0, -1.0)
    x2 = xr * xr
    s = xr * (9.9999997142e-01 + x2 * (-1.6666644356e-01 + x2 * (
        8.3328452197e-03 + x2 * (-1.9797596713e-04 + x2 * 2.5839377224e-06))))
    c = 9.9999999972e-01 + x2 * (-4.9999999217e-01 + x2 * (
        4.1666631034e-02 + x2 * (-1.3888295076e-03 + x2 * (
            2.4756776889e-05 + x2 * -2.5991777261e-07))))
    return sign * s, sign * c


def _exp(x):
    y = x * 1.4426950408889634
    i = jnp.floor(y + 0.5)
    f = y - i
    p = (1.0000000524e+00 + f * (6.9314720012e-01 + f * (
        2.4022210868e-01 + f * (5.5503405420e-02 + f * (
            9.6708152332e-03 + f * 1.3395340821e-03)))))
    ii = jnp.clip(i.astype(jnp.int32), -126, 127)
    scale = lax.bitcast_convert_type((ii + 127) << 23, jnp.float32)
    return jnp.where(x < -87.0, 0.0, scale * p)


def _synth_body(w_ref, g_ref, o_ref):
    g = g_ref[...]
    w = w_ref[...]
    p = lax.dot_general(g, w, (((1,), (0,)), ((), ())),
                        preferred_element_type=jnp.float32)
    ing = p[:, 0:8]
    outg = p[:, 8:16]
    gm = p[:, 16:24]
    dm = p[:, 24:32]
    t0c = jnp.maximum(p[:, 32:33], 1e-3)
    a_raw = p[:, 33:34]

    j = lax.broadcasted_iota(jnp.int32, (_BB, _CHUNK), 1).astype(jnp.float32)

    r = (-6.907755 / _SR) / t0c
    env_base = _exp(r * j)
    env_step = env_base[:, 64:65] * env_base[:, 64:65]
    alpha = 1.0 / (1.0 + _exp(-a_raw))

    sp = jnp.maximum(dm, 0.0) + jnp.log1p(_exp(-jnp.abs(dm)))
    k = (2.0 * jnp.pi / _SR) * (50.0 + 400.0 * sp)
    gain = ing * outg * jnp.tanh(gm)
    step_s, step_c = _sincos(k * float(_CHUNK))

    gbs, gbc = [], []
    for d in range(_N_DELAYS):
        kd = k[:, d:d + 1]
        gd = gain[:, d:d + 1]
        bs, bc = _sincos(kd * j)
        gbs.append(gd * bs)
        gbc.append(gd * bc)

    s_c = jnp.zeros((_BB, _N_DELAYS), jnp.float32)
    c_c = jnp.ones((_BB, _N_DELAYS), jnp.float32)
    env_c = jnp.ones_like(env_step)
    scs, ccs, envs = [], [], []
    for c in range(_NCHUNK):
        scs.append(s_c)
        ccs.append(c_c)
        envs.append(env_base * env_c)
        s_c, c_c = (s_c * step_c + c_c * step_s,
                    c_c * step_c - s_c * step_s)
        env_c = env_c * env_step

    late_chunks = []
    for cp in range(_NCHUNK // 2):
        c0, c1 = 2 * cp, 2 * cp + 1
        a0 = jnp.zeros((_BB, _CHUNK), jnp.float32)
        a1 = jnp.zeros((_BB, _CHUNK), jnp.float32)
        for d in range(_N_DELAYS):
            bc, bs = gbc[d], gbs[d]
            a0 = a0 + scs[c0][:, d:d + 1] * bc + ccs[c0][:, d:d + 1] * bs
            a1 = a1 + scs[c1][:, d:d + 1] * bc + ccs[c1][:, d:d + 1] * bs
        late_chunks.append(a0 * envs[c0])
        late_chunks.append(a1 * envs[c1])
    late = jnp.concatenate(late_chunks, axis=1)

    late_shift = jnp.concatenate([late[:, :1], late[:, :-1]], axis=1)
    smoothed = 0.5 * (late + late_shift)
    late = alpha * smoothed + (1.0 - alpha) * late

    early_pad = jnp.concatenate(
        [g[:, _C_EARLY:_C_EARLY + _EARLY_LEN],
         jnp.zeros((_BB, _TPAD - _EARLY_LEN), jnp.float32)], axis=1)
    out = early_pad + late
    o_ref[...] = out[:, :_IR_LEN]


def _param_matrix():
    w = np.zeros((_DP, 128), np.float32)
    for d in range(_N_DELAYS):
        w[_C_IN + d, d] = 1.0
        w[_C_OUT + d, 8 + d] = 1.0
        for q in range(4):
            w[_C_GAP + 4 * d + q, 16 + d] = 0.25
            w[_C_DAP + 4 * d + q, 24 + d] = 0.25
    w[_C_T0, 32] = 1.0
    w[_C_ALPHA, 33] = 1.0
    return jnp.asarray(w)


def _tc_synth(gathered):
    return pl.pallas_call(
        _synth_body,
        grid=(_GRID,),
        in_specs=[
            pl.BlockSpec((_DP, 128), lambda i: (0, 0)),
            pl.BlockSpec((_BB, _DP), lambda i: (i, 0)),
        ],
        out_specs=pl.BlockSpec((_BB, _IR_LEN), lambda i: (i, 0)),
        out_shape=jax.ShapeDtypeStruct((_B, _IR_LEN), jnp.float32),
    )(_param_matrix(), gathered)


def kernel(piano_model, W_input_gain, W_output_gain, W_gain_allpass,
           W_delays_allpass, W_time_rev0, W_alpha_tone, W_early_ir):
    n_instr = W_input_gain.shape[0]
    table = jnp.concatenate([
        W_early_ir,
        W_input_gain,
        W_output_gain,
        W_gain_allpass,
        W_delays_allpass,
        W_time_rev0,
        W_alpha_tone,
        jnp.zeros((n_instr, _DP - 282), jnp.float32),
    ], axis=1)
    idx = piano_model.astype(jnp.int32)
    gathered = _sc_gather(table, idx)
    return _tc_synth(gathered)


# speedup vs baseline: 1.0809x; 1.0809x over previous
import functools

import jax
import jax.numpy as jnp
import numpy as np
from jax import lax
from jax.experimental import pallas as pl
from jax.experimental.pallas import tpu as pltpu
from jax.experimental.pallas import tpu_sc as plsc

_SR = 24000
_IR_LEN = 1200
_N_DELAYS = 8
_EARLY_LEN = 200
_B = 4096

_C_EARLY = 0
_C_IN = 200
_C_OUT = 208
_C_GAP = 216
_C_DAP = 248
_C_T0 = 280
_C_ALPHA = 281
_DP = 384

_NC = 2
_NS = 16
_NW = _NC * _NS
_BPW = _B // _NW

_BB = 256
_GRID = _B // _BB


def _sc_gather(table, idx):
    mesh = plsc.VectorSubcoreMesh(core_axis_name="c", subcore_axis_name="s")

    @functools.partial(
        pl.kernel,
        mesh=mesh,
        out_type=jax.ShapeDtypeStruct((_B, _DP), jnp.float32),
        scratch_types=[
            pltpu.VMEM((_BPW,), jnp.int32),
            pltpu.VMEM((_BPW, _DP), jnp.float32),
            pltpu.SemaphoreType.DMA,
        ],
    )
    def k(table_hbm, idx_hbm, out_hbm, idx_v, rows_v, sem):
        wid = lax.axis_index("s") * _NC + lax.axis_index("c")
        base = wid * _BPW
        pltpu.sync_copy(idx_hbm.at[pl.ds(base, _BPW)], idx_v)
        pltpu.async_copy(table_hbm.at[idx_v], rows_v, sem).wait()
        pltpu.sync_copy(rows_v, out_hbm.at[pl.ds(base, _BPW)])

    return k(table, idx)


_CHUNK = 128
_NCHUNK = -(-_IR_LEN // _CHUNK)
_TPAD = _NCHUNK * _CHUNK

_PI_HI = 3.140625
_PI_LO = 9.6765358979e-04


def _sincos(x):
    m = jnp.floor(x * (1.0 / np.pi) + 0.5)
    xr = (x - m * _PI_HI) - m * _PI_LO
    sign = jnp.where((m.astype(jnp.int32) & 1) == 0, 1.0, -1.0)
    x2 = xr * xr
    s = xr * (9.9999997142e-01 + x2 * (-1.6666644356e-01 + x2 * (
        8.3328452197e-03 + x2 * (-1.9797596713e-04 + x2 * 2.5839377224e-06))))
    c = 9.9999999972e-01 + x2 * (-4.9999999217e-01 + x2 * (
        4.1666631034e-02 + x2 * (-1.3888295076e-03 + x2 * (
            2.4756776889e-05 + x2 * -2.5991777261e-07))))
    return sign * s, sign * c


def _exp(x):
    y = x * 1.4426950408889634
    i = jnp.floor(y + 0.5)
    f = y - i
    p = (1.0000000524e+00 + f * (6.9314720012e-01 + f * (
        2.4022210868e-01 + f * (5.5503405420e-02 + f * (
            9.6708152332e-03 + f * 1.3395340821e-03)))))
    ii = jnp.clip(i.astype(jnp.int32), -126, 127)
    scale = lax.bitcast_convert_type((ii + 127) << 23, jnp.float32)
    return jnp.where(x < -87.0, 0.0, scale * p)


def _synth_body(w_ref, g_ref, o_ref):
    g = g_ref[...]
    w = w_ref[...]
    p = lax.dot_general(g, w, (((1,), (0,)), ((), ())),
                        preferred_element_type=jnp.float32)
    ing = p[:, 0:8]
    outg = p[:, 8:16]
    gm = p[:, 16:24]
    dm = p[:, 24:32]
    t0c = jnp.maximum(p[:, 32:33], 1e-3)
    a_raw = p[:, 33:34]

    j = lax.broadcasted_iota(jnp.int32, (_BB, _CHUNK), 1).astype(jnp.float32)

    r = (-6.907755 / _SR) / t0c
    env_base = _exp(r * j)
    env_step = env_base[:, 64:65] * env_base[:, 64:65]
    alpha = 1.0 / (1.0 + _exp(-a_raw))

    sp = jnp.maximum(dm, 0.0) + jnp.log1p(_exp(-jnp.abs(dm)))
    k = (2.0 * jnp.pi / _SR) * (50.0 + 400.0 * sp)
    gain = ing * outg * jnp.tanh(gm)
    step_s, step_c = _sincos(k * float(_CHUNK))

    gbs, gbc = [], []
    for d in range(_N_DELAYS):
        kd = k[:, d:d + 1]
        gd = gain[:, d:d + 1]
        bs, bc = _sincos(kd * j)
        gbs.append(gd * bs)
        gbc.append(gd * bc)

    s_c = jnp.zeros((_BB, _N_DELAYS), jnp.float32)
    c_c = jnp.ones((_BB, _N_DELAYS), jnp.float32)
    dstep_s = step_s * env_step
    dstep_c = step_c * env_step
    late_chunks = []
    for c in range(_NCHUNK):
        acc = jnp.zeros((_BB, _CHUNK), jnp.float32)
        for d in range(_N_DELAYS):
            acc = acc + s_c[:, d:d + 1] * gbc[d] + c_c[:, d:d + 1] * gbs[d]
        s_c, c_c = (s_c * dstep_c + c_c * dstep_s,
                    c_c * dstep_c - s_c * dstep_s)
        late_chunks.append(acc * env_base)
    late = jnp.concatenate(late_chunks, axis=1)

    late_shift = jnp.concatenate([late[:, :1], late[:, :-1]], axis=1)
    smoothed = 0.5 * (late + late_shift)
    late = alpha * smoothed + (1.0 - alpha) * late

    early_pad = jnp.concatenate(
        [g[:, _C_EARLY:_C_EARLY + _EARLY_LEN],
         jnp.zeros((_BB, _TPAD - _EARLY_LEN), jnp.float32)], axis=1)
    out = early_pad + late
    o_ref[...] = out[:, :_IR_LEN]


def _param_matrix():
    w = np.zeros((_DP, 128), np.float32)
    for d in range(_N_DELAYS):
        w[_C_IN + d, d] = 1.0
        w[_C_OUT + d, 8 + d] = 1.0
        for q in range(4):
            w[_C_GAP + 4 * d + q, 16 + d] = 0.25
            w[_C_DAP + 4 * d + q, 24 + d] = 0.25
    w[_C_T0, 32] = 1.0
    w[_C_ALPHA, 33] = 1.0
    return jnp.asarray(w)


def _tc_synth(gathered):
    return pl.pallas_call(
        _synth_body,
        grid=(_GRID,),
        in_specs=[
            pl.BlockSpec((_DP, 128), lambda i: (0, 0)),
            pl.BlockSpec((_BB, _DP), lambda i: (i, 0)),
        ],
        out_specs=pl.BlockSpec((_BB, _IR_LEN), lambda i: (i, 0)),
        out_shape=jax.ShapeDtypeStruct((_B, _IR_LEN), jnp.float32),
    )(_param_matrix(), gathered)


def kernel(piano_model, W_input_gain, W_output_gain, W_gain_allpass,
           W_delays_allpass, W_time_rev0, W_alpha_tone, W_early_ir):
    n_instr = W_input_gain.shape[0]
    table = jnp.concatenate([
        W_early_ir,
        W_input_gain,
        W_output_gain,
        W_gain_allpass,
        W_delays_allpass,
        W_time_rev0,
        W_alpha_tone,
        jnp.zeros((n_instr, _DP - 282), jnp.float32),
    ], axis=1)
    idx = piano_model.astype(jnp.int32)
    gathered = _sc_gather(table, idx)
    return _tc_synth(gathered)


# speedup vs baseline: 1.0952x; 1.0132x over previous
import functools

import jax
import jax.numpy as jnp
import numpy as np
from jax import lax
from jax.experimental import pallas as pl
from jax.experimental.pallas import tpu as pltpu
from jax.experimental.pallas import tpu_sc as plsc

_SR = 24000
_IR_LEN = 1200
_N_DELAYS = 8
_EARLY_LEN = 200
_B = 4096

_C_EARLY = 0
_C_IN = 200
_C_OUT = 208
_C_GAP = 216
_C_DAP = 248
_C_T0 = 280
_C_ALPHA = 281
_DP = 384

_NC = 2
_NS = 16
_NW = _NC * _NS
_BPW = _B // _NW

_BB = 256
_GRID = _B // _BB


def _sc_gather(table, idx):
    mesh = plsc.VectorSubcoreMesh(core_axis_name="c", subcore_axis_name="s")

    @functools.partial(
        pl.kernel,
        mesh=mesh,
        out_type=jax.ShapeDtypeStruct((_B, _DP), jnp.float32),
        scratch_types=[
            pltpu.VMEM((_BPW,), jnp.int32),
            pltpu.VMEM((_BPW, _DP), jnp.float32),
            pltpu.SemaphoreType.DMA,
        ],
    )
    def k(table_hbm, idx_hbm, out_hbm, idx_v, rows_v, sem):
        wid = lax.axis_index("s") * _NC + lax.axis_index("c")
        base = wid * _BPW
        pltpu.sync_copy(idx_hbm.at[pl.ds(base, _BPW)], idx_v)
        pltpu.async_copy(table_hbm.at[idx_v], rows_v, sem).wait()
        pltpu.sync_copy(rows_v, out_hbm.at[pl.ds(base, _BPW)])

    return k(table, idx)


_CHUNK = 128
_NCHUNK = -(-_IR_LEN // _CHUNK)
_TPAD = _NCHUNK * _CHUNK

_PI_HI = 3.140625
_PI_LO = 9.6765358979e-04


def _sincos(x):
    m = jnp.floor(x * (1.0 / np.pi) + 0.5)
    xr = (x - m * _PI_HI) - m * _PI_LO
    sign = jnp.where((m.astype(jnp.int32) & 1) == 0, 1.0, -1.0)
    x2 = xr * xr
    s = xr * (9.9999997142e-01 + x2 * (-1.6666644356e-01 + x2 * (
        8.3328452197e-03 + x2 * (-1.9797596713e-04 + x2 * 2.5839377224e-06))))
    c = 9.9999999972e-01 + x2 * (-4.9999999217e-01 + x2 * (
        4.1666631034e-02 + x2 * (-1.3888295076e-03 + x2 * (
            2.4756776889e-05 + x2 * -2.5991777261e-07))))
    return sign * s, sign * c


def _exp(x):
    y = x * 1.4426950408889634
    i = jnp.floor(y + 0.5)
    f = y - i
    p = (1.0000000524e+00 + f * (6.9314720012e-01 + f * (
        2.4022210868e-01 + f * (5.5503405420e-02 + f * (
            9.6708152332e-03 + f * 1.3395340821e-03)))))
    ii = jnp.clip(i.astype(jnp.int32), -126, 127)
    scale = lax.bitcast_convert_type((ii + 127) << 23, jnp.float32)
    return jnp.where(x < -87.0, 0.0, scale * p)


def _synth_body(w_ref, g_ref, o_ref):
    g = g_ref[...]
    w = w_ref[...]
    p = lax.dot_general(g, w, (((1,), (0,)), ((), ())),
                        preferred_element_type=jnp.float32)
    ing = p[:, 0:8]
    outg = p[:, 8:16]
    gm = p[:, 16:24]
    dm = p[:, 24:32]
    t0c = jnp.maximum(p[:, 32:33], 1e-3)
    a_raw = p[:, 33:34]

    j = lax.broadcasted_iota(jnp.int32, (_BB, _CHUNK), 1).astype(jnp.float32)

    r = (-6.907755 / _SR) / t0c
    env_base = _exp(r * j)
    env_step = env_base[:, 64:65] * env_base[:, 64:65]
    alpha = 1.0 / (1.0 + _exp(-a_raw))

    sp = jnp.maximum(dm, 0.0) + jnp.log1p(_exp(-jnp.abs(dm)))
    k = (2.0 * jnp.pi / _SR) * (50.0 + 400.0 * sp)
    gain = ing * outg * jnp.tanh(gm)
    step_s, step_c = _sincos(k * float(_CHUNK))

    gbs, gbc = [], []
    for d in range(_N_DELAYS):
        kd = k[:, d:d + 1]
        gd = gain[:, d:d + 1]
        bs, bc = _sincos(kd * j)
        gbs.append((gd * bs).astype(jnp.bfloat16))
        gbc.append((gd * bc).astype(jnp.bfloat16))

    s_c = jnp.zeros((_BB, _N_DELAYS), jnp.float32)
    c_c = jnp.ones((_BB, _N_DELAYS), jnp.float32)
    dstep_s = step_s * env_step
    dstep_c = step_c * env_step
    late_chunks = []
    for c in range(_NCHUNK):
        acc = jnp.zeros((_BB, _CHUNK), jnp.float32)
        for d in range(_N_DELAYS):
            acc = (acc + s_c[:, d:d + 1] * gbc[d].astype(jnp.float32)
                   + c_c[:, d:d + 1] * gbs[d].astype(jnp.float32))
        s_c, c_c = (s_c * dstep_c + c_c * dstep_s,
                    c_c * dstep_c - s_c * dstep_s)
        late_chunks.append(acc * env_base)
    late = jnp.concatenate(late_chunks, axis=1)

    late_shift = jnp.concatenate([late[:, :1], late[:, :-1]], axis=1)
    smoothed = 0.5 * (late + late_shift)
    late = alpha * smoothed + (1.0 - alpha) * late

    early_pad = jnp.concatenate(
        [g[:, _C_EARLY:_C_EARLY + _EARLY_LEN],
         jnp.zeros((_BB, _TPAD - _EARLY_LEN), jnp.float32)], axis=1)
    out = early_pad + late
    o_ref[...] = out[:, :_IR_LEN]


def _param_matrix():
    w = np.zeros((_DP, 128), np.float32)
    for d in range(_N_DELAYS):
        w[_C_IN + d, d] = 1.0
        w[_C_OUT + d, 8 + d] = 1.0
        for q in range(4):
            w[_C_GAP + 4 * d + q, 16 + d] = 0.25
            w[_C_DAP + 4 * d + q, 24 + d] = 0.25
    w[_C_T0, 32] = 1.0
    w[_C_ALPHA, 33] = 1.0
    return jnp.asarray(w)


def _tc_synth(gathered):
    return pl.pallas_call(
        _synth_body,
        grid=(_GRID,),
        in_specs=[
            pl.BlockSpec((_DP, 128), lambda i: (0, 0)),
            pl.BlockSpec((_BB, _DP), lambda i: (i, 0)),
        ],
        out_specs=pl.BlockSpec((_BB, _IR_LEN), lambda i: (i, 0)),
        out_shape=jax.ShapeDtypeStruct((_B, _IR_LEN), jnp.float32),
    )(_param_matrix(), gathered)


def kernel(piano_model, W_input_gain, W_output_gain, W_gain_allpass,
           W_delays_allpass, W_time_rev0, W_alpha_tone, W_early_ir):
    n_instr = W_input_gain.shape[0]
    table = jnp.concatenate([
        W_early_ir,
        W_input_gain,
        W_output_gain,
        W_gain_allpass,
        W_delays_allpass,
        W_time_rev0,
        W_alpha_tone,
        jnp.zeros((n_instr, _DP - 282), jnp.float32),
    ], axis=1)
    idx = piano_model.astype(jnp.int32)
    gathered = _sc_gather(table, idx)
    return _tc_synth(gathered)
